# baseline (speedup 1.0000x reference)
import jax
import jax.numpy as jnp
from jax import lax
from jax.experimental import pallas as pl
from jax.experimental.pallas import tpu as pltpu

_F = 256
_L = 32
_T = _L * _L


def _body(w_ref, o_ref, p01, q, sem01, sem2):
    w0 = w_ref[0, :_L, :]
    w1 = w_ref[1, :_L, :]
    w2 = w_ref[2, :_L, :]

    k_i = lax.broadcasted_iota(jnp.int32, (_L, _T), 0)
    t_i = lax.broadcasted_iota(jnp.int32, (_L, _T), 1)
    m0 = (t_i % _L == k_i).astype(jnp.float32)
    m1 = (t_i // _L == k_i).astype(jnp.float32)
    p01[:_F, :] = lax.dot_general(w0, m0, (((0,), (0,)), ((), ())),
                                  precision=lax.Precision.HIGHEST,
                                  preferred_element_type=jnp.float32)
    p01[_F:, :] = lax.dot_general(w1, m1, (((0,), (0,)), ((), ())),
                                  precision=lax.Precision.HIGHEST,
                                  preferred_element_type=jnp.float32)
    for j in range(_L):
        pltpu.make_async_copy(p01, o_ref.at[pl.ds(0, 2 * _F), j], sem01).start()

    q[...] = jnp.broadcast_to(w2[:, :, None], (_L, _F, _T))
    for j in range(_L):
        pltpu.make_async_copy(q.at[j], o_ref.at[pl.ds(2 * _F, _F), j], sem2).start()

    for j in range(_L):
        pltpu.make_async_copy(p01, o_ref.at[pl.ds(0, 2 * _F), j], sem01).wait()
        pltpu.make_async_copy(q.at[j], o_ref.at[pl.ds(2 * _F, _F), j], sem2).wait()


def kernel(x, W0, W1, W2):
    del x
    w = jnp.stack([W0, W1, W2])
    out = pl.pallas_call(
        _body,
        in_specs=[pl.BlockSpec((3, 50, _F), lambda: (0, 0, 0))],
        out_specs=pl.BlockSpec(memory_space=pl.ANY),
        out_shape=jax.ShapeDtypeStruct((3 * _F, _L, _T), jnp.float32),
        scratch_shapes=[
            pltpu.VMEM((2 * _F, _T), jnp.float32),
            pltpu.VMEM((_L, _F, _T), jnp.float32),
            pltpu.SemaphoreType.DMA,
            pltpu.SemaphoreType.DMA,
        ],
    )(w)
    return out.reshape(3 * _F, _L, _L, _L)
